# baseline (speedup 1.0000x reference)
import functools

import jax
import jax.numpy as jnp
from jax import lax
from jax.experimental import pallas as pl
from jax.experimental.pallas import tpu as pltpu
from jax.experimental.pallas import tpu_sc as plsc

_N = 10000
_E = 160000
_D = 128
_K = 7
_NC = 2
_NS = 16
_L = 16
_NW = _NC * _NS

_C = 32
_R = 8
_TOTAL = _K * _E
_CHUNKS = -(-_TOTAL // (_NW * _C * _R)) * _R
_PER_TILE = _CHUNKS * _C
_PAD_TOTAL = _PER_TILE * _NW


def _project(x, W, b):
    bn = 2000

    def body(x_ref, w_ref, b_ref, o_ref):
        o_ref[0] = lax.dot_general(
            x_ref[...], w_ref[0], (((1,), (1,)), ((), ())),
            preferred_element_type=jnp.float32) + b_ref[0]

    return pl.pallas_call(
        body,
        grid=(_K, _N // bn),
        in_specs=[
            pl.BlockSpec((bn, _D), lambda i, n: (n, 0)),
            pl.BlockSpec((1, _D, _D), lambda i, n: (i, 0, 0)),
            pl.BlockSpec((1, 1, _D), lambda i, n: (i, 0, 0)),
        ],
        out_specs=pl.BlockSpec((1, bn, _D), lambda i, n: (i, n, 0)),
        out_shape=jax.ShapeDtypeStruct((_K, _N, _D), jnp.float32),
    )(x, W, b.reshape(_K, 1, _D))


def _edge_pass(xs, meta, vals, zeros):
    mesh = plsc.VectorSubcoreMesh(core_axis_name="c", subcore_axis_name="s")

    @functools.partial(
        pl.kernel,
        out_type=jax.ShapeDtypeStruct((_NC, _N, _D), jnp.float32),
        mesh=mesh,
        scratch_types=[
            pltpu.VMEM((_R, 2, _C), jnp.int32),
            pltpu.VMEM((_R, _C), jnp.float32),
            pltpu.VMEM((_R, _C, _D), jnp.float32),
            pltpu.VMEM_SHARED((_N, _D), jnp.float32),
        ] + [pltpu.SemaphoreType.DMA] * (3 * _R),
    )
    def k(xs_hbm, meta_hbm, vals_hbm, zeros_hbm, out_hbm, meta_v, vals_v,
          rows, acc, *sems):
        msems = sems[:_R]
        gsems = sems[_R:2 * _R]
        ssems = sems[2 * _R:]
        c = lax.axis_index("c")
        s = lax.axis_index("s")
        wid = c * _NS + s
        rpw = 624
        tail = _N - _NS * rpw
        pltpu.sync_copy(zeros_hbm.at[pl.ds(s * rpw, rpw)],
                        acc.at[pl.ds(s * rpw, rpw)])

        @pl.when(s == _NS - 1)
        def _zero_tail():
            pltpu.sync_copy(zeros_hbm.at[pl.ds(_NS * rpw, tail)],
                            acc.at[pl.ds(_NS * rpw, tail)])

        plsc.subcore_barrier()

        cbase = wid * _CHUNKS

        def start_meta(it, slot):
            pltpu.async_copy(meta_hbm.at[cbase + it], meta_v.at[slot],
                             msems[slot])
            pltpu.async_copy(vals_hbm.at[cbase + it], vals_v.at[slot],
                             msems[slot])

        def wait_meta(slot):
            pltpu.make_async_copy(meta_hbm.at[0], meta_v.at[slot],
                                  msems[slot]).wait()
            pltpu.make_async_copy(vals_hbm.at[0], vals_v.at[slot],
                                  msems[slot]).wait()

        def start_gather(slot):
            pltpu.async_copy(xs_hbm.at[meta_v.at[slot, 0]], rows.at[slot],
                             gsems[slot])

        def wait_gather(slot):
            pltpu.make_async_copy(xs_hbm.at[pl.ds(0, _C)], rows.at[slot],
                                  gsems[slot]).wait()

        def start_scatter(slot):
            pltpu.async_copy(rows.at[slot], acc.at[meta_v.at[slot, 1]],
                             ssems[slot], add=True)

        def wait_scatter(slot):
            pltpu.make_async_copy(rows.at[slot], acc.at[meta_v.at[slot, 1]],
                                  ssems[slot]).wait()

        def scale(slot):
            def scale_body(g, carry):
                vvg = vals_v[slot, pl.ds(g * _L, _L)]
                for kk in range(_L):
                    val = jnp.broadcast_to(vvg[kk], (_L,))
                    e = g * _L + kk
                    for j in range(_D // _L):
                        sl = pl.ds(j * _L, _L)
                        rows[slot, e, sl] = rows[slot, e, sl] * val
                return carry

            lax.fori_loop(0, _C // _L, scale_body, 0, unroll=2)

        for j in range(6):
            start_meta(j, j)
        for j in range(5):
            wait_meta(j)
            start_gather(j)

        def oct_body(q, carry):
            it0 = q * _R
            for u in range(_R):
                it = it0 + u
                wait_gather(u)

                @pl.when(it >= 2)
                def _drain_scatter():
                    wait_scatter((u + 6) % _R)

                @pl.when(it + 5 < _CHUNKS)
                def _prefetch_gather():
                    wait_meta((u + 5) % _R)
                    start_gather((u + 5) % _R)

                scale(u)
                start_scatter(u)

                @pl.when(it + 6 < _CHUNKS)
                def _prefetch_meta():
                    start_meta(it + 6, (u + 6) % _R)

            return carry

        lax.fori_loop(0, _CHUNKS // _R, oct_body, 0)
        wait_scatter(_R - 2)
        wait_scatter(_R - 1)
        plsc.subcore_barrier()
        pltpu.sync_copy(acc.at[pl.ds(s * rpw, rpw)],
                        out_hbm.at[c, pl.ds(s * rpw, rpw)])

        @pl.when(s == _NS - 1)
        def _out_tail():
            pltpu.sync_copy(acc.at[pl.ds(_NS * rpw, tail)],
                            out_hbm.at[c, pl.ds(_NS * rpw, tail)])

    return k(xs, meta, vals, zeros)


def _reduce(partial):
    bn = 1000

    def body(p_ref, o_ref):
        o_ref[...] = p_ref[0] + p_ref[1]

    return pl.pallas_call(
        body,
        grid=(_N // bn,),
        in_specs=[pl.BlockSpec((_NC, bn, _D), lambda n: (0, n, 0))],
        out_specs=pl.BlockSpec((bn, _D), lambda n: (n, 0)),
        out_shape=jax.ShapeDtypeStruct((_N, _D), jnp.float32),
    )(partial)


def kernel(input, adjValue, edgeOne, E_start, E_end, W, b):
    del edgeOne
    xs = _project(input, W, b).reshape(_K * _N, _D)
    offs = (jnp.arange(_K, dtype=jnp.int32) * _N)[:, None]
    gidx = (E_end[:, 1, :] + offs).reshape(-1)
    sidx = E_start[:, 1, :].reshape(-1)
    vals = adjValue.reshape(-1)
    pad = _PAD_TOTAL - _TOTAL
    gidx = jnp.concatenate([gidx, jnp.zeros((pad,), jnp.int32)])
    sidx = jnp.concatenate([sidx, jnp.zeros((pad,), jnp.int32)])
    vals = jnp.concatenate([vals, jnp.zeros((pad,), jnp.float32)])
    meta = jnp.stack([gidx.reshape(-1, _C), sidx.reshape(-1, _C)], axis=1)
    vals = vals.reshape(-1, _C)
    zeros = jnp.zeros((_N, _D), jnp.float32)
    partial = _edge_pass(xs, meta, vals, zeros)
    return _reduce(partial)


# speedup vs baseline: 1.3623x; 1.3623x over previous
import functools

import jax
import jax.numpy as jnp
from jax import lax
from jax.experimental import pallas as pl
from jax.experimental.pallas import tpu as pltpu
from jax.experimental.pallas import tpu_sc as plsc

_N = 10000
_E = 160000
_D = 128
_K = 7
_NC = 2
_NS = 16
_L = 16
_NW = _NC * _NS

_C = 32
_R = 8
_TOTAL = _K * _E
_CHUNKS = -(-_TOTAL // (_NW * _C * _R)) * _R
_PER_TILE = _CHUNKS * _C
_PAD_TOTAL = _PER_TILE * _NW


def _project(x, W, b):
    bn = 1000

    def body(x_ref, w_ref, b_ref, o_ref):
        o_ref[0] = lax.dot_general(
            x_ref[...], w_ref[0], (((1,), (1,)), ((), ())),
            preferred_element_type=jnp.float32) + b_ref[0]

    return pl.pallas_call(
        body,
        grid=(_K, _N // bn),
        in_specs=[
            pl.BlockSpec((bn, _D), lambda i, n: (n, 0)),
            pl.BlockSpec((1, _D, _D), lambda i, n: (i, 0, 0)),
            pl.BlockSpec((1, 1, _D), lambda i, n: (i, 0, 0)),
        ],
        out_specs=pl.BlockSpec((1, bn, _D), lambda i, n: (i, n, 0)),
        out_shape=jax.ShapeDtypeStruct((_K, _N, _D), jnp.float32),
    )(x, W, b.reshape(_K, 1, _D))


def _edge_pass(xs, meta, vals, zeros):
    mesh = plsc.VectorSubcoreMesh(core_axis_name="c", subcore_axis_name="s")

    @functools.partial(
        pl.kernel,
        out_type=jax.ShapeDtypeStruct((_NC, _N, _D), jnp.float32),
        mesh=mesh,
        scratch_types=[
            pltpu.VMEM((_R, 2, _C), jnp.int32),
            pltpu.VMEM((_R, _C), jnp.float32),
            pltpu.VMEM((_R, _C, _D), jnp.float32),
            pltpu.VMEM_SHARED((_N, _D), jnp.float32),
        ] + [pltpu.SemaphoreType.DMA] * (3 * _R),
    )
    def k(xs_hbm, meta_hbm, vals_hbm, zeros_hbm, out_hbm, meta_v, vals_v,
          rows, acc, *sems):
        msems = sems[:_R]
        gsems = sems[_R:2 * _R]
        ssems = sems[2 * _R:]
        c = lax.axis_index("c")
        s = lax.axis_index("s")
        wid = c * _NS + s
        rpw = 624
        tail = _N - _NS * rpw
        pltpu.sync_copy(zeros_hbm.at[pl.ds(s * rpw, rpw)],
                        acc.at[pl.ds(s * rpw, rpw)])

        @pl.when(s == _NS - 1)
        def _zero_tail():
            pltpu.sync_copy(zeros_hbm.at[pl.ds(_NS * rpw, tail)],
                            acc.at[pl.ds(_NS * rpw, tail)])

        plsc.subcore_barrier()

        cbase = wid * _CHUNKS

        def start_meta(it, slot):
            pltpu.async_copy(meta_hbm.at[cbase + it], meta_v.at[slot],
                             msems[slot])
            pltpu.async_copy(vals_hbm.at[cbase + it], vals_v.at[slot],
                             msems[slot])

        def wait_meta(slot):
            pltpu.make_async_copy(meta_hbm.at[0], meta_v.at[slot],
                                  msems[slot]).wait()
            pltpu.make_async_copy(vals_hbm.at[0], vals_v.at[slot],
                                  msems[slot]).wait()

        def start_gather(slot):
            pltpu.async_copy(xs_hbm.at[meta_v.at[slot, 0]], rows.at[slot],
                             gsems[slot])

        def wait_gather(slot):
            pltpu.make_async_copy(xs_hbm.at[pl.ds(0, _C)], rows.at[slot],
                                  gsems[slot]).wait()

        def start_scatter(slot):
            pltpu.async_copy(rows.at[slot], acc.at[meta_v.at[slot, 1]],
                             ssems[slot], add=True)

        def wait_scatter(slot):
            pltpu.make_async_copy(rows.at[slot], acc.at[meta_v.at[slot, 1]],
                                  ssems[slot]).wait()

        def scale(slot):
            def scale_body(g, carry):
                vvg = vals_v[slot, pl.ds(g * _L, _L)]
                for kk in range(_L):
                    val = jnp.broadcast_to(vvg[kk], (_L,))
                    e = g * _L + kk
                    for j in range(_D // _L):
                        sl = pl.ds(j * _L, _L)
                        rows[slot, e, sl] = rows[slot, e, sl] * val
                return carry

            lax.fori_loop(0, _C // _L, scale_body, 0, unroll=2)

        for j in range(6):
            start_meta(j, j)
        for j in range(4):
            wait_meta(j)
            start_gather(j)

        def oct_body(q, carry):
            it0 = q * _R
            for u in range(_R):
                it = it0 + u
                wait_gather(u)

                @pl.when(it >= 2)
                def _drain_scatter():
                    wait_scatter((u + 6) % _R)

                @pl.when(it + 4 < _CHUNKS)
                def _prefetch_gather():
                    wait_meta((u + 4) % _R)
                    start_gather((u + 4) % _R)

                scale(u)
                start_scatter(u)

                @pl.when(it + 6 < _CHUNKS)
                def _prefetch_meta():
                    start_meta(it + 6, (u + 6) % _R)

            return carry

        lax.fori_loop(0, _CHUNKS // _R, oct_body, 0)
        wait_scatter(_R - 2)
        wait_scatter(_R - 1)
        plsc.subcore_barrier()
        pltpu.sync_copy(acc.at[pl.ds(s * rpw, rpw)],
                        out_hbm.at[c, pl.ds(s * rpw, rpw)])

        @pl.when(s == _NS - 1)
        def _out_tail():
            pltpu.sync_copy(acc.at[pl.ds(_NS * rpw, tail)],
                            out_hbm.at[c, pl.ds(_NS * rpw, tail)])

    return k(xs, meta, vals, zeros)


def _reduce(partial):
    bn = 1000

    def body(p_ref, o_ref):
        o_ref[...] = p_ref[0] + p_ref[1]

    return pl.pallas_call(
        body,
        grid=(_N // bn,),
        in_specs=[pl.BlockSpec((_NC, bn, _D), lambda n: (0, n, 0))],
        out_specs=pl.BlockSpec((bn, _D), lambda n: (n, 0)),
        out_shape=jax.ShapeDtypeStruct((_N, _D), jnp.float32),
    )(partial)


def kernel(input, adjValue, edgeOne, E_start, E_end, W, b):
    del edgeOne
    xs = _project(input, W, b).reshape(_K * _N, _D)
    offs = (jnp.arange(_K, dtype=jnp.int32) * _N)[:, None]
    gidx = (E_end[:, 1, :] + offs).reshape(-1)
    sidx = E_start[:, 1, :].reshape(-1)
    vals = adjValue.reshape(-1)
    pad = _PAD_TOTAL - _TOTAL
    gidx = jnp.concatenate([gidx, jnp.zeros((pad,), jnp.int32)])
    sidx = jnp.concatenate([sidx, jnp.zeros((pad,), jnp.int32)])
    vals = jnp.concatenate([vals, jnp.zeros((pad,), jnp.float32)])
    meta = jnp.stack([gidx.reshape(-1, _C), sidx.reshape(-1, _C)], axis=1)
    vals = vals.reshape(-1, _C)
    zeros = jnp.zeros((_N, _D), jnp.float32)
    partial = _edge_pass(xs, meta, vals, zeros)
    return _reduce(partial)


# speedup vs baseline: 1.3922x; 1.0219x over previous
import functools

import jax
import jax.numpy as jnp
from jax import lax
from jax.experimental import pallas as pl
from jax.experimental.pallas import tpu as pltpu
from jax.experimental.pallas import tpu_sc as plsc

_N = 10000
_E = 160000
_D = 128
_K = 7
_NC = 2
_NS = 16
_L = 16
_NW = _NC * _NS

_C = 32
_R = 8
_TOTAL = _K * _E
_CHUNKS = -(-_TOTAL // (_NW * _C * _R)) * _R
_PER_TILE = _CHUNKS * _C
_PAD_TOTAL = _PER_TILE * _NW


def _project(x, W, b):
    bn = 2000

    def body(x_ref, w_ref, b_ref, o_ref):
        o_ref[0] = lax.dot_general(
            x_ref[...], w_ref[0], (((1,), (1,)), ((), ())),
            preferred_element_type=jnp.float32) + b_ref[0]

    return pl.pallas_call(
        body,
        grid=(_K, _N // bn),
        in_specs=[
            pl.BlockSpec((bn, _D), lambda i, n: (n, 0)),
            pl.BlockSpec((1, _D, _D), lambda i, n: (i, 0, 0)),
            pl.BlockSpec((1, 1, _D), lambda i, n: (i, 0, 0)),
        ],
        out_specs=pl.BlockSpec((1, bn, _D), lambda i, n: (i, n, 0)),
        out_shape=jax.ShapeDtypeStruct((_K, _N, _D), jnp.float32),
    )(x, W, b.reshape(_K, 1, _D))


def _edge_pass(xs, meta, vals, zeros):
    mesh = plsc.VectorSubcoreMesh(core_axis_name="c", subcore_axis_name="s")

    @functools.partial(
        pl.kernel,
        out_type=jax.ShapeDtypeStruct((_NC, _N, _D), jnp.float32),
        mesh=mesh,
        scratch_types=[
            pltpu.VMEM((_R, 2, _C), jnp.int32),
            pltpu.VMEM((_R, _C), jnp.float32),
            pltpu.VMEM((_R, _C, _D), jnp.float32),
            pltpu.VMEM_SHARED((_N, _D), jnp.float32),
        ] + [pltpu.SemaphoreType.DMA] * (3 * _R),
    )
    def k(xs_hbm, meta_hbm, vals_hbm, zeros_hbm, out_hbm, meta_v, vals_v,
          rows, acc, *sems):
        msems = sems[:_R]
        gsems = sems[_R:2 * _R]
        ssems = sems[2 * _R:]
        c = lax.axis_index("c")
        s = lax.axis_index("s")
        wid = c * _NS + s
        rpw = 624
        tail = _N - _NS * rpw
        pltpu.sync_copy(zeros_hbm.at[pl.ds(s * rpw, rpw)],
                        acc.at[pl.ds(s * rpw, rpw)])

        @pl.when(s == _NS - 1)
        def _zero_tail():
            pltpu.sync_copy(zeros_hbm.at[pl.ds(_NS * rpw, tail)],
                            acc.at[pl.ds(_NS * rpw, tail)])

        plsc.subcore_barrier()

        cbase = wid * _CHUNKS

        def start_meta(it, slot):
            pltpu.async_copy(meta_hbm.at[cbase + it], meta_v.at[slot],
                             msems[slot])
            pltpu.async_copy(vals_hbm.at[cbase + it], vals_v.at[slot],
                             msems[slot])

        def wait_meta(slot):
            pltpu.make_async_copy(meta_hbm.at[0], meta_v.at[slot],
                                  msems[slot]).wait()
            pltpu.make_async_copy(vals_hbm.at[0], vals_v.at[slot],
                                  msems[slot]).wait()

        def start_gather(slot):
            pltpu.async_copy(xs_hbm.at[meta_v.at[slot, 0]], rows.at[slot],
                             gsems[slot])

        def wait_gather(slot):
            pltpu.make_async_copy(xs_hbm.at[pl.ds(0, _C)], rows.at[slot],
                                  gsems[slot]).wait()

        def start_scatter(slot):
            pltpu.async_copy(rows.at[slot], acc.at[meta_v.at[slot, 1]],
                             ssems[slot], add=True)

        def wait_scatter(slot):
            pltpu.make_async_copy(rows.at[slot], acc.at[meta_v.at[slot, 1]],
                                  ssems[slot]).wait()

        def scale(slot):
            def scale_body(g, carry):
                vvg = vals_v[slot, pl.ds(g * _L, _L)]
                for kk in range(_L):
                    val = jnp.broadcast_to(vvg[kk], (_L,))
                    e = g * _L + kk
                    for j in range(_D // _L):
                        sl = pl.ds(j * _L, _L)
                        rows[slot, e, sl] = rows[slot, e, sl] * val
                return carry

            lax.fori_loop(0, _C // _L, scale_body, 0, unroll=2)

        for j in range(6):
            start_meta(j, j)
        for j in range(4):
            wait_meta(j)
            start_gather(j)

        def oct_body(q, carry):
            it0 = q * _R
            for u in range(_R):
                it = it0 + u
                wait_gather(u)

                @pl.when(it >= 2)
                def _drain_scatter():
                    wait_scatter((u + 6) % _R)

                @pl.when(it + 4 < _CHUNKS)
                def _prefetch_gather():
                    wait_meta((u + 4) % _R)
                    start_gather((u + 4) % _R)

                scale(u)
                start_scatter(u)

                @pl.when(it + 6 < _CHUNKS)
                def _prefetch_meta():
                    start_meta(it + 6, (u + 6) % _R)

            return carry

        lax.fori_loop(0, _CHUNKS // _R, oct_body, 0)
        wait_scatter(_R - 2)
        wait_scatter(_R - 1)
        plsc.subcore_barrier()
        pltpu.sync_copy(acc.at[pl.ds(s * rpw, rpw)],
                        out_hbm.at[c, pl.ds(s * rpw, rpw)])

        @pl.when(s == _NS - 1)
        def _out_tail():
            pltpu.sync_copy(acc.at[pl.ds(_NS * rpw, tail)],
                            out_hbm.at[c, pl.ds(_NS * rpw, tail)])

    return k(xs, meta, vals, zeros)


def _reduce(partial):
    bn = 1000

    def body(p_ref, o_ref):
        o_ref[...] = p_ref[0] + p_ref[1]

    return pl.pallas_call(
        body,
        grid=(_N // bn,),
        in_specs=[pl.BlockSpec((_NC, bn, _D), lambda n: (0, n, 0))],
        out_specs=pl.BlockSpec((bn, _D), lambda n: (n, 0)),
        out_shape=jax.ShapeDtypeStruct((_N, _D), jnp.float32),
    )(partial)


def kernel(input, adjValue, edgeOne, E_start, E_end, W, b):
    del edgeOne
    xs = _project(input, W, b).reshape(_K * _N, _D)
    offs = (jnp.arange(_K, dtype=jnp.int32) * _N)[:, None]
    gidx = (E_end[:, 1, :] + offs).reshape(-1)
    sidx = E_start[:, 1, :].reshape(-1)
    vals = adjValue.reshape(-1)
    pad = _PAD_TOTAL - _TOTAL
    gidx = jnp.concatenate([gidx, jnp.zeros((pad,), jnp.int32)])
    sidx = jnp.concatenate([sidx, jnp.zeros((pad,), jnp.int32)])
    vals = jnp.concatenate([vals, jnp.zeros((pad,), jnp.float32)])
    meta = jnp.stack([gidx.reshape(-1, _C), sidx.reshape(-1, _C)], axis=1)
    vals = vals.reshape(-1, _C)
    zeros = jnp.zeros((_N, _D), jnp.float32)
    partial = _edge_pass(xs, meta, vals, zeros)
    return _reduce(partial)
